# Initial kernel scaffold; baseline (speedup 1.0000x reference)
import jax
import jax.numpy as jnp
from jax.experimental import pallas as pl


def kernel(x, Wg, bg, Ws1, bs1, Ws2, bs2, We1, be1, We2, be2):
    raise NotImplementedError("write your pallas kernel here")


# speedup vs baseline: 1.1410x; 1.1410x over previous
import functools

import jax
import jax.numpy as jnp
from jax.experimental import pallas as pl
from jax.experimental.pallas import tpu as pltpu


def _router_moe_body(x_ref, wg_ref, bg_ref, w1_ref, b1_ref, w2_ref, b2_ref,
                     out_ref, w_scr):
    e = pl.program_id(1)
    i2 = pl.program_id(2)

    @pl.when(jnp.logical_and(e == 0, i2 == 0))
    def _():
        x = x_ref[...]
        logits = jnp.dot(x, wg_ref[...], preferred_element_type=jnp.float32)
        logits = logits + bg_ref[...]
        m = jnp.max(logits, axis=-1, keepdims=True)
        p = jnp.exp(logits - m)
        p = p / jnp.sum(p, axis=-1, keepdims=True)
        top1 = jnp.max(p, axis=-1, keepdims=True)
        is1 = p == top1
        p2 = jnp.where(is1, -jnp.inf, p)
        top2 = jnp.max(p2, axis=-1, keepdims=True)
        is2 = p2 == top2
        denom = top1 + top2 + 1e-9
        w_scr[...] = jnp.where(is1 | is2, p, 0.0) / denom
        out_ref[...] = jnp.zeros_like(out_ref)

    x = x_ref[...]
    h = jax.nn.gelu(
        jnp.dot(x, w1_ref[0], preferred_element_type=jnp.float32) + b1_ref[0])
    y = jnp.dot(h, w2_ref[0], preferred_element_type=jnp.float32)
    y = y + jnp.where(i2 == 0, 1.0, 0.0) * b2_ref[0]
    lane = jax.lax.broadcasted_iota(jnp.int32, w_scr.shape, 1)
    w_col = jnp.sum(jnp.where(lane == e, w_scr[...], 0.0), axis=1, keepdims=True)
    out_ref[...] += w_col * y


def _shared_body(x_ref, w1_ref, b1_ref, w2_ref, b2_ref, moe_ref, out_ref):
    i2 = pl.program_id(1)

    @pl.when(i2 == 0)
    def _():
        out_ref[...] = x_ref[...] + moe_ref[...] + b2_ref[...]

    h = jax.nn.gelu(
        jnp.dot(x_ref[...], w1_ref[...], preferred_element_type=jnp.float32)
        + b1_ref[...])
    out_ref[...] += jnp.dot(h, w2_ref[...], preferred_element_type=jnp.float32)


def _moe_dense(xs, Wg, bg, Ws1, bs1, Ws2, bs2, We1, be1, We2, be2,
               bt, ib):
    S, D = xs.shape
    E = Wg.shape[1]
    I = Ws1.shape[1]
    T = S // bt
    NI = I // ib
    moe = pl.pallas_call(
        _router_moe_body,
        grid=(T, E, NI),
        in_specs=[
            pl.BlockSpec((bt, D), lambda t, e, i: (t, 0)),
            pl.BlockSpec((D, E), lambda t, e, i: (0, 0)),
            pl.BlockSpec((1, E), lambda t, e, i: (0, 0)),
            pl.BlockSpec((1, D, ib), lambda t, e, i: (e, 0, i)),
            pl.BlockSpec((1, 1, ib), lambda t, e, i: (e, 0, i)),
            pl.BlockSpec((1, ib, D), lambda t, e, i: (e, i, 0)),
            pl.BlockSpec((1, 1, D), lambda t, e, i: (e, 0, 0)),
        ],
        out_specs=pl.BlockSpec((bt, D), lambda t, e, i: (t, 0)),
        out_shape=jax.ShapeDtypeStruct((S, D), jnp.float32),
        scratch_shapes=[pltpu.VMEM((bt, E), jnp.float32)],
    )(xs, Wg, bg.reshape(1, E), We1, be1.reshape(E, 1, I), We2,
      be2.reshape(E, 1, D))
    out = pl.pallas_call(
        _shared_body,
        grid=(T, NI),
        in_specs=[
            pl.BlockSpec((bt, D), lambda t, i: (t, 0)),
            pl.BlockSpec((D, ib), lambda t, i: (0, i)),
            pl.BlockSpec((1, ib), lambda t, i: (0, i)),
            pl.BlockSpec((ib, D), lambda t, i: (i, 0)),
            pl.BlockSpec((1, D), lambda t, i: (0, 0)),
            pl.BlockSpec((bt, D), lambda t, i: (t, 0)),
        ],
        out_specs=pl.BlockSpec((bt, D), lambda t, i: (t, 0)),
        out_shape=jax.ShapeDtypeStruct((S, D), jnp.float32),
    )(xs, Ws1, bs1.reshape(1, I), Ws2, bs2.reshape(1, D), moe)
    return out


def kernel(x, Wg, bg, Ws1, bs1, Ws2, bs2, We1, be1, We2, be2):
    B, S, D = x.shape
    xs = x.reshape(S, D)
    bt = min(S, 1024)
    ib = min(Ws1.shape[1], 512)
    out = _moe_dense(xs, Wg, bg, Ws1, bs1, Ws2, bs2, We1, be1, We2, be2, bt, ib)
    return out.reshape(B, S, D)
